# Initial kernel scaffold; baseline (speedup 1.0000x reference)
import functools

import jax
import jax.numpy as jnp
from jax import lax
from jax.experimental import pallas as pl
from jax.experimental.pallas import tpu as pltpu
from jax.experimental.pallas import tpu_sc as plsc

_BATCH = 16384
_F = 26
_BINS = 100000
_D = 64
_B = _BATCH * _F
_NW = 32
_BPW = _B // _NW
_C = 128
_T = _BPW // _C
_NBUF = 4
_LANES = 16


def _sc_gather(tab_flat, x_flat):
    mesh = plsc.VectorSubcoreMesh(core_axis_name="c", subcore_axis_name="s")

    scratch = [pltpu.VMEM((_BPW,), jnp.int32)]
    scratch += [pltpu.VMEM((_C, _D), jnp.float32) for _ in range(_NBUF)]
    scratch += [pltpu.SemaphoreType.DMA for _ in range(2 * _NBUF)]

    @functools.partial(
        pl.kernel,
        out_type=jax.ShapeDtypeStruct((_B, _D), jnp.float32),
        mesh=mesh,
        scratch_types=scratch,
    )
    def body(tab_hbm, x_hbm, out_hbm, idx_v, *rest):
        bufs = rest[:_NBUF]
        gsem = rest[_NBUF:2 * _NBUF]
        psem = rest[2 * _NBUF:]

        wid = lax.axis_index("s") * 2 + lax.axis_index("c")
        base = wid * _BPW

        pltpu.sync_copy(x_hbm.at[pl.ds(base, _BPW)], idx_v)

        @pl.loop(0, _BPW // _LANES)
        def _(i):
            off = i * _LANES
            pos = lax.iota(jnp.int32, _LANES) + off
            f = lax.rem(pos, _F)
            idx_v[pl.ds(off, _LANES)] = idx_v[pl.ds(off, _LANES)] + f * _BINS

        def start_gather(j, b):
            return pltpu.async_copy(
                tab_hbm.at[idx_v.at[pl.ds(j * _C, _C)]], bufs[b], gsem[b])

        for b in range(_NBUF):
            start_gather(b, b)

        def drain(j, b, with_next):
            pltpu.make_async_copy(
                tab_hbm.at[idx_v.at[pl.ds(j * _C, _C)]], bufs[b], gsem[b]).wait()
            pltpu.async_copy(
                bufs[b], out_hbm.at[pl.ds(base + j * _C, _C)], psem[b]).wait()
            if with_next:
                start_gather(j + _NBUF, b)

        @pl.loop(0, _T - _NBUF, step=_NBUF)
        def _(g):
            for b in range(_NBUF):
                drain(g + b, b, True)

        for b in range(_NBUF):
            drain(_T - _NBUF + b, b, False)

    return body(tab_flat, x_flat)


def kernel(x, tables):
    tab_flat = tables.reshape(_F * _BINS, _D)
    x_flat = x.reshape(_B)
    out = _sc_gather(tab_flat, x_flat)
    return out.reshape(_BATCH, _F * _D)


# speedup vs baseline: 1.1243x; 1.1243x over previous
import functools

import jax
import jax.numpy as jnp
from jax import lax
from jax.experimental import pallas as pl
from jax.experimental.pallas import tpu as pltpu
from jax.experimental.pallas import tpu_sc as plsc

_BATCH = 16384
_F = 26
_BINS = 100000
_D = 64
_B = _BATCH * _F
_NW = 32
_BPW = _B // _NW
_C = 128
_T = _BPW // _C
_NBUF = 4
_LANES = 16


def _sc_gather(tab_flat, x_flat):
    mesh = plsc.VectorSubcoreMesh(core_axis_name="c", subcore_axis_name="s")

    scratch = [pltpu.VMEM((_BPW,), jnp.int32)]
    scratch += [pltpu.VMEM((_C, _D), jnp.float32) for _ in range(_NBUF)]
    scratch += [pltpu.SemaphoreType.DMA for _ in range(2 * _NBUF)]

    @functools.partial(
        pl.kernel,
        out_type=jax.ShapeDtypeStruct((_B, _D), jnp.float32),
        mesh=mesh,
        scratch_types=scratch,
        compiler_params=pltpu.CompilerParams(use_tc_tiling_on_sc=False),
    )
    def body(tab_hbm, x_hbm, out_hbm, idx_v, *rest):
        bufs = rest[:_NBUF]
        gsem = rest[_NBUF:2 * _NBUF]
        psem = rest[2 * _NBUF:]

        wid = lax.axis_index("s") * 2 + lax.axis_index("c")
        base = wid * _BPW

        pltpu.sync_copy(x_hbm.at[pl.ds(base, _BPW)], idx_v)

        @pl.loop(0, _BPW // _LANES)
        def _(i):
            off = i * _LANES
            pos = lax.iota(jnp.int32, _LANES) + off
            f = lax.rem(pos, _F)
            idx_v[pl.ds(off, _LANES)] = idx_v[pl.ds(off, _LANES)] + f * _BINS

        def start_gather(j, b):
            return pltpu.async_copy(
                tab_hbm.at[idx_v.at[pl.ds(j * _C, _C)]], bufs[b], gsem[b])

        for b in range(_NBUF):
            start_gather(b, b)

        def drain(j, b, with_next):
            pltpu.make_async_copy(
                tab_hbm.at[idx_v.at[pl.ds(j * _C, _C)]], bufs[b], gsem[b]).wait()
            pltpu.async_copy(
                bufs[b], out_hbm.at[pl.ds(base + j * _C, _C)], psem[b]).wait()
            if with_next:
                start_gather(j + _NBUF, b)

        @pl.loop(0, _T - _NBUF, step=_NBUF)
        def _(g):
            for b in range(_NBUF):
                drain(g + b, b, True)

        for b in range(_NBUF):
            drain(_T - _NBUF + b, b, False)

    return body(tab_flat, x_flat)


def kernel(x, tables):
    tab_flat = tables.reshape(_F * _BINS, _D)
    x_flat = x.reshape(_B)
    out = _sc_gather(tab_flat, x_flat)
    return out.reshape(_BATCH, _F * _D)
